# Initial kernel scaffold; baseline (speedup 1.0000x reference)
import jax
import jax.numpy as jnp
from jax.experimental import pallas as pl


def kernel(x_nchw, weight, bias, gamma, beta):
    raise NotImplementedError("write your pallas kernel here")


# speedup vs baseline: 1.1578x; 1.1578x over previous
import functools

import jax
import jax.numpy as jnp
from jax.experimental import pallas as pl
from jax.experimental.pallas import tpu as pltpu


def _stats_kernel(x_ref, gram_ref, sum_ref):
    @pl.when(pl.program_id(0) == 0)
    def _init():
        gram_ref[...] = jnp.zeros_like(gram_ref)
        sum_ref[...] = jnp.zeros_like(sum_ref)

    x = x_ref[...]
    xb = x.astype(jnp.bfloat16)
    g = jax.lax.dot_general(
        xb, xb,
        dimension_numbers=(((2,), (2,)), ((0,), (0,))),
        preferred_element_type=jnp.float32)
    gram_ref[...] += jnp.sum(g, axis=0)
    sum_ref[...] += jnp.sum(x, axis=(0, 2))[:, None]


def _fold_kernel(gram_ref, sum_ref, w_ref, gamma_ref, beta_ref,
                 scale_ref, shift_ref, *, inv_m, eps):
    g = gram_ref[...]
    w = w_ref[...]
    mean_x = sum_ref[...] * jnp.float32(inv_m)
    a = jnp.dot(w, g, preferred_element_type=jnp.float32)
    ey2 = jnp.sum(a * w, axis=1, keepdims=True) * jnp.float32(inv_m)
    mean_y = jnp.dot(w, mean_x, preferred_element_type=jnp.float32)
    var = jnp.maximum(ey2 - mean_y * mean_y, 0.0)
    inv_std = jax.lax.rsqrt(var + jnp.float32(eps))
    scale = gamma_ref[...] * inv_std
    scale_ref[...] = scale
    shift_ref[...] = beta_ref[...] - mean_y * scale


def _apply_kernel(x_ref, w_ref, scale_ref, shift_ref, o_ref):
    w = w_ref[...]
    scale = scale_ref[...]
    shift = shift_ref[...]
    for i in range(x_ref.shape[0]):
        xb = x_ref[i].astype(jnp.bfloat16)
        y = jnp.dot(w, xb, preferred_element_type=jnp.float32)
        z = y * scale + shift
        o_ref[i] = jnp.where(z > 0, z, jnp.float32(0.2) * z)


def _conv_block1(x_nchw, weight, gamma, beta, *, eps=1e-5):
    N, C_in, H, W = x_nchw.shape
    C_out = weight.shape[0]
    HW = H * W
    M = N * HW

    x3 = x_nchw.reshape(N, C_in, HW)
    w32 = weight.reshape(C_out, C_in).astype(jnp.float32)
    wb16 = w32.astype(jnp.bfloat16)
    gamma2 = gamma.reshape(C_out, 1).astype(jnp.float32)
    beta2 = beta.reshape(C_out, 1).astype(jnp.float32)

    nb = 2 if N % 2 == 0 else 1
    steps = N // nb

    vmem_limit = 64 * 1024 * 1024
    flops_mm = 2 * M * C_in * C_out
    bytes_x = M * C_in * 4
    bytes_out = M * C_out * 4

    x_spec = pl.BlockSpec((nb, C_in, HW), lambda j: (j, 0, 0))

    gram, sums = pl.pallas_call(
        _stats_kernel,
        out_shape=(jax.ShapeDtypeStruct((C_in, C_in), jnp.float32),
                   jax.ShapeDtypeStruct((C_in, 1), jnp.float32)),
        grid=(steps,),
        in_specs=[x_spec],
        out_specs=(pl.BlockSpec((C_in, C_in), lambda j: (0, 0)),
                   pl.BlockSpec((C_in, 1), lambda j: (0, 0))),
        compiler_params=pltpu.CompilerParams(
            dimension_semantics=("arbitrary",),
            vmem_limit_bytes=vmem_limit),
        cost_estimate=pl.CostEstimate(flops=M * C_in * C_in,
                                      transcendentals=0, bytes_accessed=bytes_x),
    )(x3)

    scale, shift = pl.pallas_call(
        functools.partial(_fold_kernel, inv_m=1.0 / M, eps=float(eps)),
        out_shape=(jax.ShapeDtypeStruct((C_out, 1), jnp.float32),
                   jax.ShapeDtypeStruct((C_out, 1), jnp.float32)),
        compiler_params=pltpu.CompilerParams(vmem_limit_bytes=vmem_limit),
    )(gram, sums, w32, gamma2, beta2)

    w_spec = pl.BlockSpec((C_out, C_in), lambda j: (0, 0))
    vec_spec = pl.BlockSpec((C_out, 1), lambda j: (0, 0))
    out3 = pl.pallas_call(
        _apply_kernel,
        out_shape=jax.ShapeDtypeStruct((N, C_out, HW), jnp.float32),
        grid=(steps,),
        in_specs=[x_spec, w_spec, vec_spec, vec_spec],
        out_specs=pl.BlockSpec((nb, C_out, HW), lambda j: (j, 0, 0)),
        compiler_params=pltpu.CompilerParams(
            dimension_semantics=("arbitrary",),
            vmem_limit_bytes=vmem_limit),
        cost_estimate=pl.CostEstimate(flops=flops_mm, transcendentals=0,
                                      bytes_accessed=bytes_x + bytes_out),
    )(x3, wb16, scale, shift)

    return out3.reshape(N, C_out, H, W)


def kernel(x_nchw, weight, bias, gamma, beta):
    del bias
    return _conv_block1(x_nchw, weight, gamma, beta)


# speedup vs baseline: 1.2677x; 1.0949x over previous
import functools

import jax
import jax.numpy as jnp
from jax.experimental import pallas as pl
from jax.experimental.pallas import tpu as pltpu


def _fused_kernel(x_ref, w16_ref, w32_ref, gamma_ref, beta_ref, o_ref,
                  xres_ref, gram_ref, sum_ref, scale_ref, shift_ref,
                  *, nb, steps, inv_m, eps):
    p = pl.program_id(0)
    j = pl.program_id(1)

    @pl.when(p == 0)
    def _phase0():
        @pl.when(j == 0)
        def _init():
            gram_ref[...] = jnp.zeros_like(gram_ref)
            sum_ref[...] = jnp.zeros_like(sum_ref)

        x = x_ref[...]
        xb = x.astype(jnp.bfloat16)
        xres_ref[pl.ds(j * nb, nb)] = xb
        g = jax.lax.dot_general(
            xb, xb,
            dimension_numbers=(((2,), (2,)), ((0,), (0,))),
            preferred_element_type=jnp.float32)
        gram_ref[...] += jnp.sum(g, axis=0)
        sum_ref[...] += jnp.sum(x, axis=(0, 2))[:, None]

        @pl.when(j == steps - 1)
        def _fold():
            w = w32_ref[...]
            mean_x = sum_ref[...] * jnp.float32(inv_m)
            a = jnp.dot(w, gram_ref[...], preferred_element_type=jnp.float32)
            ey2 = jnp.sum(a * w, axis=1, keepdims=True) * jnp.float32(inv_m)
            mean_y = jnp.dot(w, mean_x, preferred_element_type=jnp.float32)
            var = jnp.maximum(ey2 - mean_y * mean_y, 0.0)
            inv_std = jax.lax.rsqrt(var + jnp.float32(eps))
            scale = gamma_ref[...] * inv_std
            scale_ref[...] = scale
            shift_ref[...] = beta_ref[...] - mean_y * scale

    @pl.when(p == 1)
    def _phase1():
        w = w16_ref[...]
        scale = scale_ref[...]
        shift = shift_ref[...]
        for i in range(nb):
            xb = xres_ref[j * nb + i]
            y = jnp.dot(w, xb, preferred_element_type=jnp.float32)
            z = y * scale + shift
            o_ref[i] = jnp.where(z > 0, z, jnp.float32(0.2) * z)


def _conv_block1(x_nchw, weight, gamma, beta, *, eps=1e-5):
    N, C_in, H, W = x_nchw.shape
    C_out = weight.shape[0]
    HW = H * W
    M = N * HW

    x3 = x_nchw.reshape(N, C_in, HW)
    w32 = weight.reshape(C_out, C_in).astype(jnp.float32)
    wb16 = w32.astype(jnp.bfloat16)
    gamma2 = gamma.reshape(C_out, 1).astype(jnp.float32)
    beta2 = beta.reshape(C_out, 1).astype(jnp.float32)

    nb = 2 if N % 2 == 0 else 1
    steps = N // nb

    flops_mm = 2 * M * C_in * C_out
    bytes_x = M * C_in * 4
    bytes_out = M * C_out * 4

    x_spec = pl.BlockSpec(
        (nb, C_in, HW),
        lambda p, j, s=steps: (jnp.where(p == 0, j, s - 1), 0, 0))
    o_spec = pl.BlockSpec(
        (nb, C_out, HW),
        lambda p, j: (jnp.where(p == 0, 0, j), 0, 0))
    const_spec = lambda shape: pl.BlockSpec(shape, lambda p, j: (0, 0))

    out3 = pl.pallas_call(
        functools.partial(_fused_kernel, nb=nb, steps=steps,
                          inv_m=1.0 / M, eps=float(eps)),
        out_shape=jax.ShapeDtypeStruct((N, C_out, HW), jnp.float32),
        grid=(2, steps),
        in_specs=[x_spec,
                  const_spec((C_out, C_in)),
                  const_spec((C_out, C_in)),
                  const_spec((C_out, 1)),
                  const_spec((C_out, 1))],
        out_specs=o_spec,
        scratch_shapes=[
            pltpu.VMEM((N, C_in, HW), jnp.bfloat16),
            pltpu.VMEM((C_in, C_in), jnp.float32),
            pltpu.VMEM((C_in, 1), jnp.float32),
            pltpu.VMEM((C_out, 1), jnp.float32),
            pltpu.VMEM((C_out, 1), jnp.float32),
        ],
        compiler_params=pltpu.CompilerParams(
            dimension_semantics=("arbitrary", "arbitrary"),
            vmem_limit_bytes=60 * 1024 * 1024),
        cost_estimate=pl.CostEstimate(flops=flops_mm + M * C_in * C_in,
                                      transcendentals=0,
                                      bytes_accessed=bytes_x + bytes_out),
    )(x3, wb16, w32, gamma2, beta2)

    return out3.reshape(N, C_out, H, W)


def kernel(x_nchw, weight, bias, gamma, beta):
    del bias
    return _conv_block1(x_nchw, weight, gamma, beta)
